# baseline (speedup 1.0000x reference)
import functools

import jax
import jax.numpy as jnp
from jax import lax
from jax.experimental import pallas as pl
from jax.experimental.pallas import tpu as pltpu
from jax.experimental.pallas import tpu_sc as plsc

NC, NS, LANES = 2, 16, 16
NW = NC * NS

BATCH, SEQ, DIM = 4, 8192, 1024
ROWS_PER_W = SEQ // NW
R = 8
NBLK = ROWS_PER_W // R
DEPTH = 3
UNROLL = 8
NB_MAIN = (NBLK // DEPTH) * DEPTH


def _sc_body(x_hbm, w_hbm, o_hbm, *refs):
    xbufs = [refs[q * BATCH:(q + 1) * BATCH] for q in range(DEPTH)]
    wq = refs[DEPTH * BATCH:DEPTH * BATCH + DEPTH]
    sems = refs[DEPTH * BATCH + DEPTH:]
    sins = [sems[q * BATCH:(q + 1) * BATCH] for q in range(DEPTH)]
    souts = [sems[(DEPTH + q) * BATCH:(DEPTH + q + 1) * BATCH]
             for q in range(DEPTH)]
    swq = sems[2 * DEPTH * BATCH:]

    wid = lax.axis_index("s") * NC + lax.axis_index("c")
    row0 = wid * ROWS_PER_W

    def wslice(blk):
        return w_hbm.at[pl.ds(row0 + blk * R, R)]

    def xslice(ref, blk, b):
        return ref.at[pl.ds(b * SEQ + row0 + blk * R, R)]

    def add_rows(wbuf, xbuf):
        @plsc.parallel_loop(0, R)
        def _rows(r):
            @plsc.parallel_loop(0, DIM, step=LANES, unroll=UNROLL)
            def _cols(c):
                wv = wbuf[r, pl.ds(c, LANES)]
                plsc.addupdate(xbuf.at[r, pl.ds(c, LANES)], wv)

    def half(blk, q, tail=False):
        qn = (q + 2) % DEPTH
        pltpu.make_async_copy(wslice(blk), wq[q], swq[q]).wait()
        for b in range(BATCH):
            pltpu.make_async_copy(xslice(x_hbm, blk, b), xbufs[q][b],
                                  sins[q][b]).wait()
            add_rows(wq[q], xbufs[q][b])
            pltpu.async_copy(xbufs[q][b], xslice(o_hbm, blk, b), souts[q][b])

        if tail:
            return

        @pl.when(blk + 2 < NBLK)
        def _prep():
            pltpu.async_copy(wslice(blk + 2), wq[qn], swq[qn])
            for b in range(BATCH):
                @pl.when(blk > 0)
                def _drain():
                    pltpu.make_async_copy(xbufs[qn][b],
                                          xslice(o_hbm, blk - 1, b),
                                          souts[qn][b]).wait()
                pltpu.async_copy(xslice(x_hbm, blk + 2, b), xbufs[qn][b],
                                sins[qn][b])

    for blk in (0, 1):
        pltpu.async_copy(wslice(blk), wq[blk], swq[blk])
        for b in range(BATCH):
            pltpu.async_copy(xslice(x_hbm, blk, b), xbufs[blk][b],
                             sins[blk][b])

    def body(i, _):
        blk = DEPTH * i
        half(blk, 0)
        half(blk + 1, 1)
        half(blk + 2, 2)
        return _

    lax.fori_loop(0, NB_MAIN // DEPTH, body, 0)
    for blk in range(NB_MAIN, NBLK):
        half(blk, blk % DEPTH, tail=True)
    for blk in range(NBLK - DEPTH, NBLK):
        q = blk % DEPTH
        for b in range(BATCH):
            pltpu.make_async_copy(xbufs[q][b], xslice(o_hbm, blk, b),
                                  souts[q][b]).wait()


@functools.partial(jax.jit, static_argnums=())
def kernel(inputs, W):
    batch, seq_len, dim = inputs.shape
    run = pl.kernel(
        _sc_body,
        out_type=jax.ShapeDtypeStruct((batch * seq_len, dim), inputs.dtype),
        mesh=plsc.VectorSubcoreMesh(core_axis_name="c", subcore_axis_name="s"),
        compiler_params=pltpu.CompilerParams(use_tc_tiling_on_sc=True),
        scratch_types=(
            [pltpu.VMEM((R, DIM), jnp.float32)] * (DEPTH * BATCH + DEPTH)
            + [pltpu.SemaphoreType.DMA] * (2 * DEPTH * BATCH + DEPTH)
        ),
    )
    out = run(inputs.reshape(batch * seq_len, dim), W)
    return out.reshape(batch, seq_len, dim)
